# baseline (speedup 1.0000x reference)
import functools

import jax
import jax.numpy as jnp
from jax import lax
from jax.experimental import pallas as pl
from jax.experimental.pallas import tpu as pltpu
from jax.experimental.pallas import tpu_sc as plsc

BATCH = 4096
SEQ = 200
HIDDEN = 64
PAD = 2 * HIDDEN
VOCAB = 1000000
NUM_TOKENS = BATCH * SEQ
NUM_WORKERS = 32
B_PER_W = BATCH // NUM_WORKERS
G = 2
ROWS_G = G * SEQ
NG = B_PER_W // G
C0, C1 = 128, SEQ - 128

VSPAN = 256
NFULL = VOCAB // VSPAN
VREM = VOCAB - NFULL * VSPAN
TAILW = 128
NSLAB = NFULL + 1
SLAB_ITERS = -(-NSLAB // NUM_WORKERS)


def _transpose_body(table_t, tail_t, tp_hbm, in_a, in_b, ou_a, ou_b, isems,
                    osems):
    wid = lax.axis_index("s") * 2 + lax.axis_index("c")
    lanes = [lax.iota(jnp.int32, 16) + 16 * g for g in range(4)]

    def in_parts(s, buf, width):
        if width == VREM:
            return tail_t, buf.at[:, pl.ds(0, TAILW)]
        return table_t.at[:, pl.ds(s * VSPAN, width)], buf

    def out_slice(s, width):
        return tp_hbm.at[pl.ds(s * VSPAN, width), :]

    def load(s, buf, sem, width):
        src, dst = in_parts(s, buf, width)
        pltpu.async_copy(src, dst, sem)

    def load_wait(s, buf, sem, width):
        src, dst = in_parts(s, buf, width)
        pltpu.make_async_copy(src, dst, sem).wait()

    def transpose(ibuf, obuf, width):
        def body(i, carry):
            for u in range(2):
                v = 2 * i + u
                cols = jnp.full((16,), v, jnp.int32)
                for g in range(4):
                    vals = plsc.load_gather(ibuf, [lanes[g], cols])
                    obuf[v, pl.ds(16 * g, 16)] = vals
            return carry

        lax.fori_loop(0, width // 2, body, 0)

    def store(s, buf, sem, width):
        pltpu.async_copy(buf.at[pl.ds(0, width), :], out_slice(s, width), sem)

    def store_wait(s, buf, sem, width):
        pltpu.make_async_copy(
            buf.at[pl.ds(0, width), :], out_slice(s, width), sem
        ).wait()

    def slab(i):
        return wid + i * NUM_WORKERS

    def handle(s, ibuf, isem, obuf, osem):
        @pl.when(s < NFULL)
        def _():
            load_wait(s, ibuf, isem, VSPAN)
            transpose(ibuf, obuf, VSPAN)
            store(s, obuf, osem, VSPAN)

        @pl.when(s == NFULL)
        def _():
            load_wait(s, ibuf, isem, VREM)
            transpose(ibuf, obuf, VREM)
            store(s, obuf, osem, VREM)

    def start(s, ibuf, isem):
        @pl.when(s < NFULL)
        def _():
            load(s, ibuf, isem, VSPAN)

        @pl.when(s == NFULL)
        def _():
            load(s, ibuf, isem, VREM)

    def finish(s, obuf, osem):
        @pl.when(s < NFULL)
        def _():
            store_wait(s, obuf, osem, VSPAN)

        @pl.when(s == NFULL)
        def _():
            store_wait(s, obuf, osem, VREM)

    start(slab(0), in_a, isems.at[0])

    def body(i, carry):
        s = slab(i)

        @pl.when(lax.rem(i, 2) == 0)
        def _():
            start(slab(i + 1), in_b, isems.at[1])
            pl.when(i >= 2)(lambda: finish(slab(i - 2), ou_a, osems.at[0]))
            handle(s, in_a, isems.at[0], ou_a, osems.at[0])

        @pl.when(lax.rem(i, 2) == 1)
        def _():
            start(slab(i + 1), in_a, isems.at[0])
            pl.when(i >= 2)(lambda: finish(slab(i - 2), ou_b, osems.at[1]))
            handle(s, in_b, isems.at[1], ou_b, osems.at[1])

        return carry

    lax.fori_loop(0, SLAB_ITERS, body, 0)
    assert SLAB_ITERS % 2 == 1
    finish(slab(SLAB_ITERS - 2), ou_b, osems.at[1])
    finish(slab(SLAB_ITERS - 1), ou_a, osems.at[0])


def _gather_body(tok_hbm, table_hbm, out_hbm, idx_v, buf_a, buf_b, gsem_a,
                 gsem_b, osem_a, osem_b):
    wid = lax.axis_index("s") * 2 + lax.axis_index("c")
    t0 = wid * B_PER_W * SEQ
    pltpu.sync_copy(tok_hbm.at[pl.ds(t0, B_PER_W * SEQ)], idx_v)

    def fire(g, buf, gsem):
        for j in range(G):
            off = (g * G + j) * SEQ
            pltpu.async_copy(
                table_hbm.at[idx_v.at[pl.ds(off, C0)]],
                buf.at[pl.ds(j * SEQ, C0)],
                gsem,
            )
            pltpu.async_copy(
                table_hbm.at[idx_v.at[pl.ds(off + C0, C1)]],
                buf.at[pl.ds(j * SEQ + C0, C1)],
                gsem,
            )

    def drain(buf, gsem):
        for j in range(G):
            pltpu.make_async_copy(
                table_hbm.at[idx_v.at[pl.ds(0, C0)]],
                buf.at[pl.ds(j * SEQ, C0)],
                gsem,
            ).wait()
            pltpu.make_async_copy(
                table_hbm.at[idx_v.at[pl.ds(0, C1)]],
                buf.at[pl.ds(j * SEQ + C0, C1)],
                gsem,
            ).wait()

    def out_slice(g):
        return out_hbm.at[pl.ds(t0 + g * ROWS_G, ROWS_G), :]

    def store(g, buf, osem):
        pltpu.async_copy(buf, out_slice(g), osem)

    def store_wait(g, buf, osem):
        pltpu.make_async_copy(buf, out_slice(g), osem).wait()

    fire(0, buf_a, gsem_a)
    fire(1, buf_b, gsem_b)
    drain(buf_a, gsem_a)
    store(0, buf_a, osem_a)

    def body(i, carry):
        store_wait(2 * i, buf_a, osem_a)
        fire(2 * i + 2, buf_a, gsem_a)
        drain(buf_b, gsem_b)
        store(2 * i + 1, buf_b, osem_b)
        store_wait(2 * i + 1, buf_b, osem_b)
        fire(2 * i + 3, buf_b, gsem_b)
        drain(buf_a, gsem_a)
        store(2 * i + 2, buf_a, osem_a)
        return carry

    lax.fori_loop(0, (NG - 2) // 2, body, 0)

    drain(buf_b, gsem_b)
    store(NG - 1, buf_b, osem_b)
    store_wait(NG - 2, buf_a, osem_a)
    store_wait(NG - 1, buf_b, osem_b)


@jax.jit
def _embed(tokens, table_t, tail_t):
    mesh = plsc.VectorSubcoreMesh(core_axis_name="c", subcore_axis_name="s")

    table_p = functools.partial(
        pl.kernel,
        mesh=mesh,
        compiler_params=pltpu.CompilerParams(needs_layout_passes=False),
        out_type=jax.ShapeDtypeStruct((VOCAB, PAD), jnp.float32),
        scratch_types=[
            pltpu.VMEM((HIDDEN, VSPAN), jnp.float32),
            pltpu.VMEM((HIDDEN, VSPAN), jnp.float32),
            pltpu.VMEM((VSPAN, PAD), jnp.float32),
            pltpu.VMEM((VSPAN, PAD), jnp.float32),
            pltpu.SemaphoreType.DMA((2,)),
            pltpu.SemaphoreType.DMA((2,)),
        ],
    )(_transpose_body)(table_t, tail_t)

    out = functools.partial(
        pl.kernel,
        mesh=mesh,
        compiler_params=pltpu.CompilerParams(use_tc_tiling_on_sc=False),
        out_type=jax.ShapeDtypeStruct((NUM_TOKENS, PAD), jnp.float32),
        scratch_types=[
            pltpu.VMEM((B_PER_W * SEQ,), jnp.int32),
            pltpu.VMEM((ROWS_G, PAD), jnp.float32),
            pltpu.VMEM((ROWS_G, PAD), jnp.float32),
            pltpu.SemaphoreType.DMA,
            pltpu.SemaphoreType.DMA,
            pltpu.SemaphoreType.DMA,
            pltpu.SemaphoreType.DMA,
        ],
    )(_gather_body)(tokens, table_p)
    return out


def kernel(tokens, embedding_weight):
    table_t = embedding_weight.T
    tail_t = jnp.pad(table_t[:, NFULL * VSPAN:], ((0, 0), (0, TAILW - VREM)))
    out = _embed(tokens.astype(jnp.int32).ravel(), table_t, tail_t)
    return out.reshape(BATCH, SEQ, PAD)[..., :HIDDEN]


# speedup vs baseline: 2.1322x; 2.1322x over previous
import functools

import jax
import jax.numpy as jnp
from jax import lax
from jax.experimental import pallas as pl
from jax.experimental.pallas import tpu as pltpu
from jax.experimental.pallas import tpu_sc as plsc

BATCH = 4096
SEQ = 200
HIDDEN = 64
PAD = 2 * HIDDEN
NUM_TOKENS = BATCH * SEQ
NUM_WORKERS = 32
B_PER_W = BATCH // NUM_WORKERS
G = 2
ROWS_G = G * SEQ
NG = B_PER_W // G
C0, C1 = 128, SEQ - 128


@jax.jit
def _embed(tokens, table):
    mesh = plsc.VectorSubcoreMesh(core_axis_name="c", subcore_axis_name="s")

    @functools.partial(
        pl.kernel,
        mesh=mesh,
        compiler_params=pltpu.CompilerParams(use_tc_tiling_on_sc=False),
        out_type=jax.ShapeDtypeStruct((NUM_TOKENS, PAD), jnp.float32),
        scratch_types=[
            pltpu.VMEM((B_PER_W * SEQ,), jnp.int32),
            pltpu.VMEM((ROWS_G, HIDDEN), jnp.float32),
            pltpu.VMEM((ROWS_G, HIDDEN), jnp.float32),
            pltpu.SemaphoreType.DMA,
            pltpu.SemaphoreType.DMA,
            pltpu.SemaphoreType.DMA,
            pltpu.SemaphoreType.DMA,
        ],
    )
    def k(tok_hbm, table_hbm, out_hbm, idx_v, buf_a, buf_b, gsem_a, gsem_b,
          osem_a, osem_b):
        wid = lax.axis_index("s") * 2 + lax.axis_index("c")
        t0 = wid * B_PER_W * SEQ
        pltpu.sync_copy(tok_hbm.at[pl.ds(t0, B_PER_W * SEQ)], idx_v)

        def fire(g, buf, gsem):
            for j in range(G):
                off = (g * G + j) * SEQ
                pltpu.async_copy(
                    table_hbm.at[idx_v.at[pl.ds(off, C0)]],
                    buf.at[pl.ds(j * SEQ, C0)],
                    gsem,
                )
                pltpu.async_copy(
                    table_hbm.at[idx_v.at[pl.ds(off + C0, C1)]],
                    buf.at[pl.ds(j * SEQ + C0, C1)],
                    gsem,
                )

        def drain(buf, gsem):
            for j in range(G):
                pltpu.make_async_copy(
                    table_hbm.at[idx_v.at[pl.ds(0, C0)]],
                    buf.at[pl.ds(j * SEQ, C0)],
                    gsem,
                ).wait()
                pltpu.make_async_copy(
                    table_hbm.at[idx_v.at[pl.ds(0, C1)]],
                    buf.at[pl.ds(j * SEQ + C0, C1)],
                    gsem,
                ).wait()

        def out_slice(g):
            return out_hbm.at[pl.ds(t0 + g * ROWS_G, ROWS_G), pl.ds(0, HIDDEN)]

        def store(g, buf, osem):
            pltpu.async_copy(buf, out_slice(g), osem)

        def store_wait(g, buf, osem):
            pltpu.make_async_copy(buf, out_slice(g), osem).wait()

        fire(0, buf_a, gsem_a)
        fire(1, buf_b, gsem_b)
        drain(buf_a, gsem_a)
        store(0, buf_a, osem_a)

        def body(i, carry):
            store_wait(2 * i, buf_a, osem_a)
            fire(2 * i + 2, buf_a, gsem_a)
            drain(buf_b, gsem_b)
            store(2 * i + 1, buf_b, osem_b)
            store_wait(2 * i + 1, buf_b, osem_b)
            fire(2 * i + 3, buf_b, gsem_b)
            drain(buf_a, gsem_a)
            store(2 * i + 2, buf_a, osem_a)
            return carry

        lax.fori_loop(0, (NG - 2) // 2, body, 0)

        drain(buf_b, gsem_b)
        store(NG - 1, buf_b, osem_b)
        store_wait(NG - 2, buf_a, osem_a)
        store_wait(NG - 1, buf_b, osem_b)

    return k(tokens, table)


def kernel(tokens, embedding_weight):
    out = _embed(tokens.astype(jnp.int32).ravel(), embedding_weight)
    return out.reshape(BATCH, SEQ, PAD)[..., :HIDDEN]
